# baseline (speedup 1.0000x reference)
import jax
import jax.numpy as jnp
from jax.experimental import pallas as pl
from jax.experimental.pallas import tpu as pltpu

IN_CHANNELS = 256
HIDDEN_DIM = 128
NUM_EXPERTS = 4
BB = 2
HSPLIT = 2


def _gate_kernel(x_ref, w1_ref, b1_ref, w2_ref, b2_ref, out_ref, acc_ref):
    g = pl.program_id(0)
    j = pl.program_id(1)
    hw = x_ref.shape[1] * HSPLIT * x_ref.shape[2]
    parts = []
    for i in range(BB):
        parts.append(jnp.sum(x_ref[i], axis=0))

    @pl.when(j == 0)
    def _():
        for i in range(BB):
            acc_ref[i] = parts[i]

    @pl.when(j == HSPLIT - 1)
    def _():
        rows = []
        for i in range(BB):
            tot = acc_ref[i] + parts[i]
            rows.append(jnp.sum(tot, axis=0))
        pooled = jnp.stack(rows, axis=0) * (1.0 / hw)
        h = jnp.dot(pooled, w1_ref[...], preferred_element_type=jnp.float32)
        h = h + b1_ref[...]
        h = 0.5 * h * (1.0 + jax.lax.erf(h * 0.7071067811865476))
        logits = jnp.dot(h, w2_ref[...], preferred_element_type=jnp.float32)
        logits = logits + b2_ref[...]
        m = jnp.max(logits, axis=-1, keepdims=True)
        e = jnp.exp(logits - m)
        out_ref[pl.ds(g * BB, BB), :] = e / jnp.sum(e, axis=-1, keepdims=True)


@jax.jit
def kernel(img_emb, W1, b1, W2, b2):
    B, C, H, W = img_emb.shape
    x = img_emb.transpose(0, 2, 3, 1)
    b1r = b1.reshape(1, HIDDEN_DIM)
    b2r = b2.reshape(1, NUM_EXPERTS)
    Hh = H // HSPLIT
    out = pl.pallas_call(
        _gate_kernel,
        grid=(B // BB, HSPLIT),
        in_specs=[
            pl.BlockSpec((BB, Hh, W, C), lambda g, j: (g, j, 0, 0)),
            pl.BlockSpec((C, HIDDEN_DIM), lambda g, j: (0, 0)),
            pl.BlockSpec((1, HIDDEN_DIM), lambda g, j: (0, 0)),
            pl.BlockSpec((HIDDEN_DIM, NUM_EXPERTS), lambda g, j: (0, 0)),
            pl.BlockSpec((1, NUM_EXPERTS), lambda g, j: (0, 0)),
        ],
        out_specs=pl.BlockSpec((B, NUM_EXPERTS), lambda g, j: (0, 0)),
        out_shape=jax.ShapeDtypeStruct((B, NUM_EXPERTS), jnp.float32),
        scratch_shapes=[pltpu.VMEM((BB, W, C), jnp.float32)],
    )(x, W1, b1r, W2, b2r)
    return out
